# Initial kernel scaffold; baseline (speedup 1.0000x reference)
import jax
import jax.numpy as jnp
from jax.experimental import pallas as pl


def kernel(inputs, vote_weights):
    raise NotImplementedError("write your pallas kernel here")


# speedup vs baseline: 2.9913x; 2.9913x over previous
import functools

import jax
import jax.numpy as jnp
from jax.experimental import pallas as pl
from jax.experimental.pallas import tpu as pltpu

_B_BLK = 2048


def _vote_body(w_ref, x_ref, o_ref):
    xf = x_ref[...].astype(jnp.float32)
    c1 = jnp.zeros((_B_BLK,), jnp.float32)
    tot = jnp.float32(0.0)
    for m in range(26):
        wm = w_ref[m]
        c1 = c1 + wm * xf[m, :]
        tot = tot + wm
    o_ref[...] = (c1 + c1 > tot).astype(jnp.int32)


def kernel(inputs, vote_weights):
    n_models, batch = inputs.shape
    grid = (batch // _B_BLK,)
    wi = jax.lax.bitcast_convert_type(vote_weights, jnp.uint32)
    wr = (wi + jnp.uint32(0x8000) + ((wi >> 16) & jnp.uint32(1))) & jnp.uint32(0xFFFF0000)
    vote_weights = jax.lax.bitcast_convert_type(wr, jnp.float32)
    return pl.pallas_call(
        _vote_body,
        grid=grid,
        in_specs=[
            pl.BlockSpec(memory_space=pltpu.SMEM),
            pl.BlockSpec((n_models, _B_BLK), lambda i: (0, i)),
        ],
        out_specs=pl.BlockSpec((_B_BLK,), lambda i: (i,)),
        out_shape=jax.ShapeDtypeStruct((batch,), jnp.int32),
    )(vote_weights, inputs)
